# baseline (speedup 1.0000x reference)
import functools

import jax
import jax.numpy as jnp
from jax import lax
from jax.experimental import pallas as pl
from jax.experimental.pallas import tpu as pltpu
from jax.experimental.pallas import tpu_sc as plsc

NUM_DOMAINS = 4
N = 32768
D = 512

NC = 2
NS = 16
NW = NC * NS
N_SC = 20480
N_TC = N - N_SC
ROWS_PER_W = N_SC // NW
CHUNK = 64
NCHUNK = ROWS_PER_W // CHUNK
REP = 4
ACC = REP * NUM_DOMAINS * D

CNT_W = 128
_PROC = True


def _sc_segment_sums(x, d, zacc):
    mesh = plsc.VectorSubcoreMesh(core_axis_name="c", subcore_axis_name="s")

    @functools.partial(
        pl.kernel,
        out_type=jax.ShapeDtypeStruct((NW, ACC), jnp.float32),
        mesh=mesh,
        compiler_params=pltpu.CompilerParams(needs_layout_passes=False),
        scratch_types=[
            pltpu.VMEM((2, CHUNK, D), jnp.float32),
            pltpu.VMEM((2, CHUNK), jnp.int32),
            pltpu.VMEM((ACC,), jnp.float32),
            pltpu.SemaphoreType.DMA,
            pltpu.SemaphoreType.DMA,
        ],
    )
    def run(x_hbm, d_hbm, zacc_hbm, psum_hbm, xbufs, dbufs, acc, sem0, sem1):
        c = lax.axis_index("c")
        s = lax.axis_index("s")
        wid = s * NC + c
        base = wid * ROWS_PER_W
        sems = (sem0, sem1)

        pltpu.sync_copy(zacc_hbm, acc)

        def issue(g, b):
            off = base + g * CHUNK
            pltpu.async_copy(x_hbm.at[pl.ds(off, CHUNK), :], xbufs.at[b],
                             sems[b])
            pltpu.async_copy(d_hbm.at[pl.ds(off, CHUNK)], dbufs.at[b],
                             sems[b])

        def wait(g, b):
            off = base + g * CHUNK
            pltpu.make_async_copy(x_hbm.at[pl.ds(off, CHUNK), :],
                                  xbufs.at[b], sems[b]).wait()
            pltpu.make_async_copy(d_hbm.at[pl.ds(off, CHUNK)],
                                  dbufs.at[b], sems[b]).wait()

        lanes = lax.iota(jnp.int32, 16)

        def process(b):
            xbuf = xbufs.at[b]
            dbuf = dbufs.at[b]

            @plsc.parallel_loop(0, CHUNK // REP, unroll=2)
            def rows4(gi):
                r0 = gi * REP
                bases = []
                for u in range(REP):
                    dom = plsc.load_gather(dbuf, [jnp.full((16,), r0 + u,
                                                           jnp.int32)])
                    bases.append(dom * D + (lanes + u * (NUM_DOMAINS * D)))
                for j in range(D // 16):
                    accj = acc.at[pl.ds(16 * j, ACC - 16 * (D // 16 - 1))]
                    for u in range(REP):
                        xv = xbuf[r0 + u, pl.ds(16 * j, 16)]
                        plsc.addupdate_scatter(accj, [bases[u]], xv)

        issue(0, 0)
        issue(1, 1)

        def pair(i, carry):
            g = 2 * i
            wait(g, 0)
            _PROC and process(0)

            @pl.when(g + 2 < NCHUNK)
            def _():
                issue(g + 2, 0)

            wait(g + 1, 1)
            _PROC and process(1)

            @pl.when(g + 3 < NCHUNK)
            def _():
                issue(g + 3, 1)

            return carry

        lax.fori_loop(0, NCHUNK // 2, pair, 0)
        pltpu.sync_copy(acc, psum_hbm.at[wid])

    return run(x, d, zacc)


def _tc_counts(d2):
    def body(d_ref, o_ref):
        dv = d_ref[...]
        for k in range(NUM_DOMAINS):
            s_k = jnp.sum((dv == k).astype(jnp.float32))
            o_ref[k:k + 1, :] = jnp.full((1, CNT_W), s_k, jnp.float32)

    return pl.pallas_call(
        body,
        out_shape=jax.ShapeDtypeStruct((NUM_DOMAINS, CNT_W), jnp.float32),
    )(d2)


BLKP = 2048


def _tc_psum(x_tc, d3p):
    def body(d_ref, x_ref, o_ref):
        i = pl.program_id(0)
        dvec = d_ref[0, 0, :]
        oh = (dvec[:, None] == lax.broadcasted_iota(
            jnp.int32, (BLKP, NUM_DOMAINS), 1)).astype(jnp.float32)
        part = lax.dot_general(oh, x_ref[...], (((0,), (0,)), ((), ())),
                               preferred_element_type=jnp.float32)

        @pl.when(i == 0)
        def _():
            o_ref[...] = part

        @pl.when(i > 0)
        def _():
            o_ref[...] += part

    return pl.pallas_call(
        body,
        grid=(N_TC // BLKP,),
        in_specs=[
            pl.BlockSpec((1, 1, BLKP), lambda i: (i, 0, 0)),
            pl.BlockSpec((BLKP, D), lambda i: (i, 0)),
        ],
        out_specs=pl.BlockSpec((NUM_DOMAINS, D), lambda i: (0, 0)),
        out_shape=jax.ShapeDtypeStruct((NUM_DOMAINS, D), jnp.float32),
    )(d3p, x_tc)


def _tc_prepare(ps3, tsum, pcnt, mean2):
    def body(ps_ref, ts_ref, pc_ref, m_ref, adj_ref):
        sums = jnp.sum(ps_ref[...], axis=0) + ts_ref[...]
        cnt = jnp.max(pc_ref[...], axis=1, keepdims=True)
        adj_ref[...] = m_ref[...] - sums / jnp.maximum(cnt, 1.0)

    return pl.pallas_call(
        body,
        out_shape=jax.ShapeDtypeStruct((NUM_DOMAINS, D), jnp.float32),
    )(ps3, tsum, pcnt, mean2)


BLK = 4096


def _tc_normalize(x, d3, adj):
    def body(d_ref, adj_ref, x_ref, o_ref):
        dvec = d_ref[0, 0, :]
        oh = (dvec[:, None] == lax.broadcasted_iota(
            jnp.int32, (BLK, NUM_DOMAINS), 1)).astype(jnp.float32)
        o_ref[...] = x_ref[...] + jnp.dot(
            oh, adj_ref[...], preferred_element_type=jnp.float32)

    return pl.pallas_call(
        body,
        grid=(N // BLK,),
        in_specs=[
            pl.BlockSpec((1, 1, BLK), lambda i: (i, 0, 0)),
            pl.BlockSpec((NUM_DOMAINS, D), lambda i: (0, 0)),
            pl.BlockSpec((BLK, D), lambda i: (i, 0)),
        ],
        out_specs=pl.BlockSpec((BLK, D), lambda i: (i, 0)),
        out_shape=jax.ShapeDtypeStruct((N, D), jnp.float32),
    )(d3, adj, x)


def kernel(X, d, mean):
    zacc = jnp.zeros((ACC,), jnp.float32)
    psum = _sc_segment_sums(X, d, zacc)
    d3p = d[N_SC:].reshape(N_TC // BLKP, 1, BLKP)
    tsum = _tc_psum(X[N_SC:], d3p)
    pcnt = _tc_counts(d.reshape(N // CNT_W, CNT_W))
    ps3 = psum.reshape(NW * REP, NUM_DOMAINS, D)
    adj = _tc_prepare(ps3, tsum, pcnt, mean.reshape(1, D))
    d3 = d.reshape(N // BLK, 1, BLK)
    return _tc_normalize(X, d3, adj)


# speedup vs baseline: 1.0504x; 1.0504x over previous
import functools

import jax
import jax.numpy as jnp
from jax import lax
from jax.experimental import pallas as pl
from jax.experimental.pallas import tpu as pltpu
from jax.experimental.pallas import tpu_sc as plsc

NUM_DOMAINS = 4
N = 32768
D = 512

NC = 2
NS = 16
NW = NC * NS
N_SC = 16384
N_TC = N - N_SC
ROWS_PER_W = N_SC // NW
CHUNK = 64
NCHUNK = ROWS_PER_W // CHUNK
REP = 4
ACC = REP * NUM_DOMAINS * D

CNT_W = 128
_PROC = True


def _sc_segment_sums(x, d):
    mesh = plsc.VectorSubcoreMesh(core_axis_name="c", subcore_axis_name="s")

    @functools.partial(
        pl.kernel,
        out_type=jax.ShapeDtypeStruct((NW, ACC), jnp.float32),
        mesh=mesh,
        compiler_params=pltpu.CompilerParams(needs_layout_passes=False),
        scratch_types=[
            pltpu.VMEM((2, CHUNK, D), jnp.float32),
            pltpu.VMEM((2, CHUNK), jnp.int32),
            pltpu.VMEM((ACC,), jnp.float32),
            pltpu.SemaphoreType.DMA,
            pltpu.SemaphoreType.DMA,
        ],
    )
    def run(x_hbm, d_hbm, psum_hbm, xbufs, dbufs, acc, sem0, sem1):
        c = lax.axis_index("c")
        s = lax.axis_index("s")
        wid = s * NC + c
        base = wid * ROWS_PER_W
        sems = (sem0, sem1)

        zero16 = jnp.zeros((16,), jnp.float32)

        @plsc.parallel_loop(0, ACC // 16, unroll=4)
        def _zero(zi):
            acc[pl.ds(zi * 16, 16)] = zero16

        def issue(g, b):
            off = base + g * CHUNK
            pltpu.async_copy(x_hbm.at[pl.ds(off, CHUNK), :], xbufs.at[b],
                             sems[b])
            pltpu.async_copy(d_hbm.at[pl.ds(off, CHUNK)], dbufs.at[b],
                             sems[b])

        def wait(g, b):
            off = base + g * CHUNK
            pltpu.make_async_copy(x_hbm.at[pl.ds(off, CHUNK), :],
                                  xbufs.at[b], sems[b]).wait()
            pltpu.make_async_copy(d_hbm.at[pl.ds(off, CHUNK)],
                                  dbufs.at[b], sems[b]).wait()

        lanes = lax.iota(jnp.int32, 16)

        def process(b):
            xbuf = xbufs.at[b]
            dbuf = dbufs.at[b]

            @plsc.parallel_loop(0, CHUNK // REP, unroll=2)
            def rows4(gi):
                r0 = gi * REP
                bases = []
                for u in range(REP):
                    dom = plsc.load_gather(dbuf, [jnp.full((16,), r0 + u,
                                                           jnp.int32)])
                    bases.append(dom * D + (lanes + u * (NUM_DOMAINS * D)))
                for j in range(D // 16):
                    accj = acc.at[pl.ds(16 * j, ACC - 16 * (D // 16 - 1))]
                    for u in range(REP):
                        xv = xbuf[r0 + u, pl.ds(16 * j, 16)]
                        plsc.addupdate_scatter(accj, [bases[u]], xv)

        issue(0, 0)
        issue(1, 1)

        def pair(i, carry):
            g = 2 * i
            wait(g, 0)
            _PROC and process(0)

            @pl.when(g + 2 < NCHUNK)
            def _():
                issue(g + 2, 0)

            wait(g + 1, 1)
            _PROC and process(1)

            @pl.when(g + 3 < NCHUNK)
            def _():
                issue(g + 3, 1)

            return carry

        lax.fori_loop(0, NCHUNK // 2, pair, 0)
        pltpu.sync_copy(acc, psum_hbm.at[wid])

    return run(x, d)


def _tc_counts(d2):
    def body(d_ref, o_ref):
        dv = d_ref[...]
        for k in range(NUM_DOMAINS):
            s_k = jnp.sum((dv == k).astype(jnp.float32))
            o_ref[k:k + 1, :] = jnp.full((1, CNT_W), s_k, jnp.float32)

    return pl.pallas_call(
        body,
        out_shape=jax.ShapeDtypeStruct((NUM_DOMAINS, CNT_W), jnp.float32),
    )(d2)


BLKP = 2048


def _tc_psum(x_tc, d3p):
    def body(d_ref, x_ref, o_ref):
        i = pl.program_id(0)
        dvec = d_ref[0, 0, :]
        oh = (dvec[:, None] == lax.broadcasted_iota(
            jnp.int32, (BLKP, NUM_DOMAINS), 1)).astype(jnp.float32)
        part = lax.dot_general(oh, x_ref[...], (((0,), (0,)), ((), ())),
                               preferred_element_type=jnp.float32)

        @pl.when(i == 0)
        def _():
            o_ref[...] = part

        @pl.when(i > 0)
        def _():
            o_ref[...] += part

    return pl.pallas_call(
        body,
        grid=(N_TC // BLKP,),
        in_specs=[
            pl.BlockSpec((1, 1, BLKP), lambda i: (i, 0, 0)),
            pl.BlockSpec((BLKP, D), lambda i: (i, 0)),
        ],
        out_specs=pl.BlockSpec((NUM_DOMAINS, D), lambda i: (0, 0)),
        out_shape=jax.ShapeDtypeStruct((NUM_DOMAINS, D), jnp.float32),
    )(d3p, x_tc)


def _tc_prepare(ps3, tsum, pcnt, mean2):
    def body(ps_ref, ts_ref, pc_ref, m_ref, adj_ref):
        sums = jnp.sum(ps_ref[...], axis=0) + ts_ref[...]
        cnt = jnp.max(pc_ref[...], axis=1, keepdims=True)
        adj_ref[...] = m_ref[...] - sums / jnp.maximum(cnt, 1.0)

    return pl.pallas_call(
        body,
        out_shape=jax.ShapeDtypeStruct((NUM_DOMAINS, D), jnp.float32),
    )(ps3, tsum, pcnt, mean2)


BLK = 4096


def _tc_normalize(x, d3, adj):
    def body(d_ref, adj_ref, x_ref, o_ref):
        dvec = d_ref[0, 0, :]
        oh = (dvec[:, None] == lax.broadcasted_iota(
            jnp.int32, (BLK, NUM_DOMAINS), 1)).astype(jnp.float32)
        o_ref[...] = x_ref[...] + jnp.dot(
            oh, adj_ref[...], preferred_element_type=jnp.float32)

    return pl.pallas_call(
        body,
        grid=(N // BLK,),
        in_specs=[
            pl.BlockSpec((1, 1, BLK), lambda i: (i, 0, 0)),
            pl.BlockSpec((NUM_DOMAINS, D), lambda i: (0, 0)),
            pl.BlockSpec((BLK, D), lambda i: (i, 0)),
        ],
        out_specs=pl.BlockSpec((BLK, D), lambda i: (i, 0)),
        out_shape=jax.ShapeDtypeStruct((N, D), jnp.float32),
    )(d3, adj, x)


def kernel(X, d, mean):
    psum = _sc_segment_sums(X, d)
    d3p = d[N_SC:].reshape(N_TC // BLKP, 1, BLKP)
    tsum = _tc_psum(X[N_SC:], d3p)
    pcnt = _tc_counts(d.reshape(N // CNT_W, CNT_W))
    ps3 = psum.reshape(NW * REP, NUM_DOMAINS, D)
    adj = _tc_prepare(ps3, tsum, pcnt, mean.reshape(1, D))
    d3 = d.reshape(N // BLK, 1, BLK)
    return _tc_normalize(X, d3, adj)
